# Initial kernel scaffold; baseline (speedup 1.0000x reference)
import jax
import jax.numpy as jnp
from jax.experimental import pallas as pl


def kernel(x, edge_index, W, b):
    raise NotImplementedError("write your pallas kernel here")


# speedup vs baseline: 7.4683x; 7.4683x over previous
import functools

import jax
import jax.numpy as jnp
from jax import lax
from jax.experimental import pallas as pl
from jax.experimental.pallas import tpu as pltpu
from jax.experimental.pallas import tpu_sc as plsc

N_NODES = 10000
N_EDGES = 320000
D = 128

NC = 2
NS = 16
NW = NC * NS
EPW = N_EDGES // NW
K = 80
NCHUNK = EPW // K
RPT = 624
TAIL = N_NODES - NS * RPT
TAIL_RB = NS * RPT


def _sc_scatter_call(x, src, dst, zrows):
    mesh = plsc.VectorSubcoreMesh(core_axis_name="c", subcore_axis_name="s")

    @functools.partial(
        pl.kernel,
        out_type=jax.ShapeDtypeStruct((NC, N_NODES, D), jnp.float32),
        mesh=mesh,
        scratch_types=[
            pltpu.VMEM((EPW,), jnp.int32),
            pltpu.VMEM((EPW,), jnp.int32),
            pltpu.VMEM((K,), jnp.int32),
            pltpu.VMEM((K,), jnp.int32),
            pltpu.VMEM((K, D), jnp.float32),
            pltpu.VMEM_SHARED((N_NODES, D), jnp.float32),
            pltpu.SemaphoreType.DMA,
        ],
    )
    def k(x_hbm, src_hbm, dst_hbm, z_hbm, out_hbm,
          src_all, dst_all, src_v, dst_v, rows_v, acc, sem):
        cid = lax.axis_index("c")
        sid = lax.axis_index("s")
        wid = sid * NC + cid
        tb = wid * EPW
        rb = sid * RPT

        pltpu.sync_copy(z_hbm.at[pl.ds(0, RPT)], acc.at[pl.ds(rb, RPT)])

        @pl.when(sid == 0)
        def _zero_tail():
            pltpu.sync_copy(z_hbm.at[pl.ds(0, TAIL)],
                            acc.at[pl.ds(TAIL_RB, TAIL)])

        pltpu.sync_copy(src_hbm.at[pl.ds(tb, EPW)], src_all)
        pltpu.sync_copy(dst_hbm.at[pl.ds(tb, EPW)], dst_all)
        plsc.subcore_barrier()

        def chunk_body(ci, carry):
            off = ci * K
            for j in range(K // 16):
                src_v[pl.ds(j * 16, 16)] = src_all[pl.ds(off + j * 16, 16)]
                dst_v[pl.ds(j * 16, 16)] = dst_all[pl.ds(off + j * 16, 16)]
            pltpu.async_copy(x_hbm.at[src_v], rows_v, sem).wait()
            pltpu.sync_copy(rows_v, acc.at[dst_v], add=True)
            return carry

        lax.fori_loop(0, NCHUNK, chunk_body, 0)
        plsc.subcore_barrier()

        pltpu.sync_copy(acc.at[pl.ds(rb, RPT)], out_hbm.at[cid, pl.ds(rb, RPT)])

        @pl.when(sid == 0)
        def _write_tail():
            pltpu.sync_copy(acc.at[pl.ds(TAIL_RB, TAIL)],
                            out_hbm.at[cid, pl.ds(TAIL_RB, TAIL)])

    return k(x, src, dst, zrows)


def _tc_linear_call(p0, p1, wt, b2d):
    R = 2000

    def mm_body(p0_ref, p1_ref, wt_ref, b_ref, o_ref):
        s = p0_ref[...] + p1_ref[...]
        o_ref[...] = (
            jnp.dot(s, wt_ref[...], preferred_element_type=jnp.float32)
            + b_ref[...]
        )

    return pl.pallas_call(
        mm_body,
        grid=(N_NODES // R,),
        in_specs=[
            pl.BlockSpec((R, D), lambda i: (i, 0)),
            pl.BlockSpec((R, D), lambda i: (i, 0)),
            pl.BlockSpec((D, D), lambda i: (0, 0)),
            pl.BlockSpec((1, D), lambda i: (0, 0)),
        ],
        out_specs=pl.BlockSpec((R, D), lambda i: (i, 0)),
        out_shape=jax.ShapeDtypeStruct((N_NODES, D), jnp.float32),
    )(p0, p1, wt, b2d)


def kernel(x, edge_index, W, b):
    src = edge_index[0].astype(jnp.int32)
    dst = edge_index[1].astype(jnp.int32)
    zrows = jnp.zeros((RPT, D), jnp.float32)
    partials = _sc_scatter_call(x, src, dst, zrows)
    return _tc_linear_call(partials[0], partials[1], W.T, b.reshape(1, D))


# speedup vs baseline: 11.5280x; 1.5436x over previous
import functools

import jax
import jax.numpy as jnp
from jax import lax
from jax.experimental import pallas as pl
from jax.experimental.pallas import tpu as pltpu
from jax.experimental.pallas import tpu_sc as plsc

N_NODES = 10000
N_EDGES = 320000
D = 128

NC = 2
NS = 16
NW = NC * NS
EPW = N_EDGES // NW
K = 80
NCHUNK = EPW // K
RPT = 624
TAIL = N_NODES - NS * RPT
TAIL_RB = NS * RPT


def _sc_scatter_call(x, src, dst, zrows):
    mesh = plsc.VectorSubcoreMesh(core_axis_name="c", subcore_axis_name="s")

    @functools.partial(
        pl.kernel,
        out_type=jax.ShapeDtypeStruct((NC, N_NODES, D), jnp.float32),
        mesh=mesh,
        scratch_types=[
            pltpu.VMEM((EPW,), jnp.int32),
            pltpu.VMEM((EPW,), jnp.int32),
            pltpu.VMEM((K,), jnp.int32),
            pltpu.VMEM((K,), jnp.int32),
            pltpu.VMEM((K, D), jnp.float32),
            pltpu.VMEM((K,), jnp.int32),
            pltpu.VMEM((K,), jnp.int32),
            pltpu.VMEM((K, D), jnp.float32),
            pltpu.VMEM_SHARED((N_NODES, D), jnp.float32),
            pltpu.SemaphoreType.DMA,
            pltpu.SemaphoreType.DMA,
        ],
    )
    def k(x_hbm, src_hbm, dst_hbm, z_hbm, out_hbm,
          src_all, dst_all,
          src_v0, dst_v0, rows_v0,
          src_v1, dst_v1, rows_v1,
          acc, sem0, sem1):
        cid = lax.axis_index("c")
        sid = lax.axis_index("s")
        wid = sid * NC + cid
        tb = wid * EPW
        rb = sid * RPT

        pltpu.sync_copy(z_hbm.at[pl.ds(0, RPT)], acc.at[pl.ds(rb, RPT)])

        @pl.when(sid == 0)
        def _zero_tail():
            pltpu.sync_copy(z_hbm.at[pl.ds(0, TAIL)],
                            acc.at[pl.ds(TAIL_RB, TAIL)])

        pltpu.sync_copy(src_hbm.at[pl.ds(tb, EPW)], src_all)
        pltpu.sync_copy(dst_hbm.at[pl.ds(tb, EPW)], dst_all)
        plsc.subcore_barrier()

        def copy_idx(off, sbuf, dbuf):
            for j in range(K // 16):
                sbuf[pl.ds(j * 16, 16)] = src_all[pl.ds(off + j * 16, 16)]
                dbuf[pl.ds(j * 16, 16)] = dst_all[pl.ds(off + j * 16, 16)]

        copy_idx(0, src_v0, dst_v0)
        pltpu.async_copy(x_hbm.at[src_v0], rows_v0, sem0)

        def pair_body(p, carry):
            copy_idx((2 * p + 1) * K, src_v1, dst_v1)
            pltpu.async_copy(x_hbm.at[src_v1], rows_v1, sem1)
            pltpu.make_async_copy(x_hbm.at[src_v0], rows_v0, sem0).wait()
            pltpu.sync_copy(rows_v0, acc.at[dst_v0], add=True)

            @pl.when(2 * p + 2 < NCHUNK)
            def _():
                copy_idx((2 * p + 2) * K, src_v0, dst_v0)
                pltpu.async_copy(x_hbm.at[src_v0], rows_v0, sem0)

            pltpu.make_async_copy(x_hbm.at[src_v1], rows_v1, sem1).wait()
            pltpu.sync_copy(rows_v1, acc.at[dst_v1], add=True)
            return carry

        lax.fori_loop(0, NCHUNK // 2, pair_body, 0)

        pltpu.make_async_copy(x_hbm.at[src_v0], rows_v0, sem0).wait()
        pltpu.sync_copy(rows_v0, acc.at[dst_v0], add=True)

        plsc.subcore_barrier()

        pltpu.sync_copy(acc.at[pl.ds(rb, RPT)], out_hbm.at[cid, pl.ds(rb, RPT)])

        @pl.when(sid == 0)
        def _write_tail():
            pltpu.sync_copy(acc.at[pl.ds(TAIL_RB, TAIL)],
                            out_hbm.at[cid, pl.ds(TAIL_RB, TAIL)])

    return k(x, src, dst, zrows)


def _tc_linear_call(p0, p1, wt, b2d):
    R = 2000

    def mm_body(p0_ref, p1_ref, wt_ref, b_ref, o_ref):
        s = p0_ref[...] + p1_ref[...]
        o_ref[...] = (
            jnp.dot(s, wt_ref[...], preferred_element_type=jnp.float32)
            + b_ref[...]
        )

    return pl.pallas_call(
        mm_body,
        grid=(N_NODES // R,),
        in_specs=[
            pl.BlockSpec((R, D), lambda i: (i, 0)),
            pl.BlockSpec((R, D), lambda i: (i, 0)),
            pl.BlockSpec((D, D), lambda i: (0, 0)),
            pl.BlockSpec((1, D), lambda i: (0, 0)),
        ],
        out_specs=pl.BlockSpec((R, D), lambda i: (i, 0)),
        out_shape=jax.ShapeDtypeStruct((N_NODES, D), jnp.float32),
    )(p0, p1, wt, b2d)


def kernel(x, edge_index, W, b):
    src = edge_index[0].astype(jnp.int32)
    dst = edge_index[1].astype(jnp.int32)
    zrows = jnp.zeros((RPT, D), jnp.float32)
    partials = _sc_scatter_call(x, src, dst, zrows)
    return _tc_linear_call(partials[0], partials[1], W.T, b.reshape(1, D))
